# baseline (speedup 1.0000x reference)
import functools

import jax
import jax.numpy as jnp
from jax import lax
from jax.experimental import pallas as pl
from jax.experimental.pallas import tpu as pltpu
from jax.experimental.pallas import tpu_sc as plsc

BATCH = 16384
FACTOR = 32
NROWS = 1000000
RBLK = 16384
NBLK = (NROWS + RBLK - 1) // RBLK
MROWS = NBLK * (RBLK // 4)

NUM_CORES = 2
NUM_SUBCORES = 16
NUM_WORKERS = NUM_CORES * NUM_SUBCORES
BPW = BATCH // NUM_WORKERS
CHUNK = 128
HALF = BPW // 2
NCH = HALF // CHUNK
LANES = 16


def _tc_retile_body(t_ref, m_ref):
  ii = lax.broadcasted_iota(jnp.int32, (128, 128), 0)
  jj = lax.broadcasted_iota(jnp.int32, (128, 128), 1)
  eye = jnp.where(ii == jj, 1.0, 0.0).astype(jnp.float32)
  for t in range(RBLK // 512):
    for a in range(4):
      p = 4 * t + a
      xp = t_ref[:, pl.ds(p * 128, 128)]
      yp = lax.dot_general(
          eye, xp, (((1,), (1,)), ((), ())),
          preferred_element_type=jnp.float32)
      m_ref[pl.ds(t * 128, 128), pl.ds(a * FACTOR, FACTOR)] = yp


def _tc_retile(tabT):
  return pl.pallas_call(
      _tc_retile_body,
      grid=(NBLK,),
      in_specs=[pl.BlockSpec((FACTOR, RBLK), lambda j: (0, j))],
      out_specs=pl.BlockSpec((RBLK // 4, 128), lambda j: (j, 0)),
      out_shape=jax.ShapeDtypeStruct((MROWS, 128), jnp.float32),
  )(tabT)


def _sc_gather_body(idx_hbm, m_hbm, out_hbm, idx_v, buf_a, buf_b, sem):
  wid = lax.axis_index("s") * NUM_CORES + lax.axis_index("c")
  base = wid * BPW
  nch_tab = BPW // CHUNK
  for j in range(nch_tab):
    pltpu.sync_copy(idx_hbm.at[pl.ds(base + j * CHUNK, CHUNK)],
                    idx_v.at[j])
  for j in range(nch_tab):
    for v in range(CHUNK // LANES):
      sl = pl.ds(v * LANES, LANES)
      r = idx_v[j, sl]
      idx_v[j, sl] = (
          lax.shift_left(lax.shift_right_logical(r, 14), 12)
          + lax.shift_left(lax.shift_right_logical(r, 9) & 31, 7)
          + (r & 127))

  def gather(half_idx, buf):
    cps = []
    for j in range(NCH):
      cps.append(pltpu.async_copy(
          m_hbm.at[idx_v.at[half_idx * NCH + j]],
          buf.at[pl.ds(j * CHUNK, CHUNK)], sem))
    return cps

  def drain(cps):
    for c in cps:
      c.wait()

  cps = gather(0, buf_a)
  drain(cps)
  cps = gather(1, buf_b)
  pltpu.sync_copy(buf_a, out_hbm.at[pl.ds(base, HALF)])
  drain(cps)
  pltpu.sync_copy(buf_b, out_hbm.at[pl.ds(base + HALF, HALF)])


_sc_gather = functools.partial(
    pl.kernel,
    out_type=jax.ShapeDtypeStruct((BATCH, 128), jnp.float32),
    mesh=plsc.VectorSubcoreMesh(core_axis_name="c", subcore_axis_name="s"),
    scratch_types=[
        pltpu.VMEM((BPW // CHUNK, CHUNK), jnp.int32),
        pltpu.VMEM((HALF, 128), jnp.float32),
        pltpu.VMEM((HALF, 128), jnp.float32),
        pltpu.SemaphoreType.DMA,
    ],
)(_sc_gather_body)


TC_BLK = 2048


def _tc_affine_body(u_ref, i_ref, uidx_ref, iidx_ref, w_ref, b_ref, o_ref):
  qu = lax.shift_right_logical(uidx_ref[...], 7) & 3
  qi = lax.shift_right_logical(iidx_ref[...], 7) & 3
  u128 = u_ref[...]
  i128 = i_ref[...]
  uq = jnp.zeros((TC_BLK, FACTOR), jnp.float32)
  iq = jnp.zeros((TC_BLK, FACTOR), jnp.float32)
  for q in range(4):
    sl = slice(q * FACTOR, (q + 1) * FACTOR)
    uq = uq + jnp.where(qu == q, u128[:, sl], 0.0)
    iq = iq + jnp.where(qi == q, i128[:, sl], 0.0)
  s = jnp.sum(uq * iq * w_ref[...], axis=1) + b_ref[0, 0]
  o_ref[...] = jax.nn.sigmoid(s)


def _tc_affine(u_rows, i_rows, uidx, iidx, affine_w, affine_b):
  grid = (BATCH // TC_BLK,)
  return pl.pallas_call(
      _tc_affine_body,
      grid=grid,
      in_specs=[
          pl.BlockSpec((TC_BLK, 128), lambda i: (i, 0)),
          pl.BlockSpec((TC_BLK, 128), lambda i: (i, 0)),
          pl.BlockSpec((TC_BLK, 1), lambda i: (i, 0)),
          pl.BlockSpec((TC_BLK, 1), lambda i: (i, 0)),
          pl.BlockSpec((1, FACTOR), lambda i: (0, 0)),
          pl.BlockSpec(memory_space=pltpu.SMEM),
      ],
      out_specs=pl.BlockSpec((TC_BLK,), lambda i: (i,)),
      out_shape=jax.ShapeDtypeStruct((BATCH,), jnp.float32),
  )(u_rows, i_rows, uidx.reshape(BATCH, 1), iidx.reshape(BATCH, 1),
    affine_w, affine_b.reshape(1, 1))


def kernel(user_indices, item_indices, user_table, item_table,
           affine_w, affine_b):
  uidx = user_indices.astype(jnp.int32)
  iidx = item_indices.astype(jnp.int32)
  um = _tc_retile(jnp.transpose(user_table))
  u_rows = _sc_gather(uidx, um)
  im = _tc_retile(jnp.transpose(item_table))
  i_rows = _sc_gather(iidx, im)
  return _tc_affine(u_rows, i_rows, uidx, iidx, affine_w, affine_b)


# speedup vs baseline: 1.1165x; 1.1165x over previous
import functools

import jax
import jax.numpy as jnp
from jax import lax
from jax.experimental import pallas as pl
from jax.experimental.pallas import tpu as pltpu
from jax.experimental.pallas import tpu_sc as plsc

BATCH = 16384
FACTOR = 32
NROWS = 1000000
RBLK = 32768
NBLK = (NROWS + RBLK - 1) // RBLK
MROWS = NBLK * (RBLK // 4)

NUM_CORES = 2
NUM_SUBCORES = 16
NUM_WORKERS = NUM_CORES * NUM_SUBCORES
BPW = BATCH // NUM_WORKERS
CHUNK = 128
HALF = BPW // 2
NCH = HALF // CHUNK
LANES = 16


def _tc_retile_body(t_ref, m_ref):
  ii = lax.broadcasted_iota(jnp.int32, (128, 128), 0)
  jj = lax.broadcasted_iota(jnp.int32, (128, 128), 1)
  eye = jnp.where(ii == jj, 1.0, 0.0).astype(jnp.float32)
  for t in range(RBLK // 512):
    for a in range(4):
      p = 4 * t + a
      xp = t_ref[:, pl.ds(p * 128, 128)]
      yp = lax.dot_general(
          eye, xp, (((1,), (1,)), ((), ())),
          preferred_element_type=jnp.float32)
      m_ref[pl.ds(t * 128, 128), pl.ds(a * FACTOR, FACTOR)] = yp


def _tc_retile(tabT):
  return pl.pallas_call(
      _tc_retile_body,
      grid=(NBLK,),
      in_specs=[pl.BlockSpec((FACTOR, RBLK), lambda j: (0, j))],
      out_specs=pl.BlockSpec((RBLK // 4, 128), lambda j: (j, 0)),
      out_shape=jax.ShapeDtypeStruct((MROWS, 128), jnp.float32),
  )(tabT)


def _sc_gather_body(idx_hbm, m_hbm, out_hbm, idx_v, buf_a, buf_b, sem):
  wid = lax.axis_index("s") * NUM_CORES + lax.axis_index("c")
  base = wid * BPW
  nch_tab = BPW // CHUNK
  for j in range(nch_tab):
    pltpu.sync_copy(idx_hbm.at[pl.ds(base + j * CHUNK, CHUNK)],
                    idx_v.at[j])
  for j in range(nch_tab):
    for v in range(CHUNK // LANES):
      sl = pl.ds(v * LANES, LANES)
      r = idx_v[j, sl]
      idx_v[j, sl] = (
          lax.shift_left(lax.shift_right_logical(r, 15), 13)
          + lax.shift_left(lax.shift_right_logical(r, 9) & 63, 7)
          + (r & 127))

  def gather(half_idx, buf):
    cps = []
    for j in range(NCH):
      cps.append(pltpu.async_copy(
          m_hbm.at[idx_v.at[half_idx * NCH + j]],
          buf.at[pl.ds(j * CHUNK, CHUNK)], sem))
    return cps

  def drain(cps):
    for c in cps:
      c.wait()

  cps = gather(0, buf_a)
  drain(cps)
  cps = gather(1, buf_b)
  pltpu.sync_copy(buf_a, out_hbm.at[pl.ds(base, HALF)])
  drain(cps)
  pltpu.sync_copy(buf_b, out_hbm.at[pl.ds(base + HALF, HALF)])


_sc_gather = functools.partial(
    pl.kernel,
    out_type=jax.ShapeDtypeStruct((BATCH, 128), jnp.float32),
    mesh=plsc.VectorSubcoreMesh(core_axis_name="c", subcore_axis_name="s"),
    scratch_types=[
        pltpu.VMEM((BPW // CHUNK, CHUNK), jnp.int32),
        pltpu.VMEM((HALF, 128), jnp.float32),
        pltpu.VMEM((HALF, 128), jnp.float32),
        pltpu.SemaphoreType.DMA,
    ],
)(_sc_gather_body)


TC_BLK = 2048


def _tc_affine_body(u_ref, i_ref, uidx_ref, iidx_ref, w_ref, b_ref, o_ref):
  qu = lax.shift_right_logical(uidx_ref[...], 7) & 3
  qi = lax.shift_right_logical(iidx_ref[...], 7) & 3
  u128 = u_ref[...]
  i128 = i_ref[...]
  uq = jnp.zeros((TC_BLK, FACTOR), jnp.float32)
  iq = jnp.zeros((TC_BLK, FACTOR), jnp.float32)
  for q in range(4):
    sl = slice(q * FACTOR, (q + 1) * FACTOR)
    uq = uq + jnp.where(qu == q, u128[:, sl], 0.0)
    iq = iq + jnp.where(qi == q, i128[:, sl], 0.0)
  s = jnp.sum(uq * iq * w_ref[...], axis=1) + b_ref[0, 0]
  o_ref[...] = jax.nn.sigmoid(s)


def _tc_affine(u_rows, i_rows, uidx, iidx, affine_w, affine_b):
  grid = (BATCH // TC_BLK,)
  return pl.pallas_call(
      _tc_affine_body,
      grid=grid,
      in_specs=[
          pl.BlockSpec((TC_BLK, 128), lambda i: (i, 0)),
          pl.BlockSpec((TC_BLK, 128), lambda i: (i, 0)),
          pl.BlockSpec((TC_BLK, 1), lambda i: (i, 0)),
          pl.BlockSpec((TC_BLK, 1), lambda i: (i, 0)),
          pl.BlockSpec((1, FACTOR), lambda i: (0, 0)),
          pl.BlockSpec(memory_space=pltpu.SMEM),
      ],
      out_specs=pl.BlockSpec((TC_BLK,), lambda i: (i,)),
      out_shape=jax.ShapeDtypeStruct((BATCH,), jnp.float32),
  )(u_rows, i_rows, uidx.reshape(BATCH, 1), iidx.reshape(BATCH, 1),
    affine_w, affine_b.reshape(1, 1))


def kernel(user_indices, item_indices, user_table, item_table,
           affine_w, affine_b):
  uidx = user_indices.astype(jnp.int32)
  iidx = item_indices.astype(jnp.int32)
  um = _tc_retile(jnp.transpose(user_table))
  u_rows = _sc_gather(uidx, um)
  im = _tc_retile(jnp.transpose(item_table))
  i_rows = _sc_gather(iidx, im)
  return _tc_affine(u_rows, i_rows, uidx, iidx, affine_w, affine_b)


# speedup vs baseline: 1.1412x; 1.0221x over previous
import functools

import jax
import jax.numpy as jnp
from jax import lax
from jax.experimental import pallas as pl
from jax.experimental.pallas import tpu as pltpu
from jax.experimental.pallas import tpu_sc as plsc

BATCH = 16384
FACTOR = 32
NROWS = 1000000
RBLK = 65536
NBLK = (NROWS + RBLK - 1) // RBLK
MROWS = NBLK * (RBLK // 4)

NUM_CORES = 2
NUM_SUBCORES = 16
NUM_WORKERS = NUM_CORES * NUM_SUBCORES
BPW = BATCH // NUM_WORKERS
CHUNK = 128
HALF = BPW // 2
NCH = HALF // CHUNK
LANES = 16


def _tc_retile_body(t_ref, m_ref):
  ii = lax.broadcasted_iota(jnp.int32, (128, 128), 0)
  jj = lax.broadcasted_iota(jnp.int32, (128, 128), 1)
  eye = jnp.where(ii == jj, 1.0, 0.0).astype(jnp.float32)
  for t in range(RBLK // 512):
    for a in range(4):
      p = 4 * t + a
      xp = t_ref[:, pl.ds(p * 128, 128)]
      yp = lax.dot_general(
          eye, xp, (((1,), (1,)), ((), ())),
          preferred_element_type=jnp.float32)
      m_ref[pl.ds(t * 128, 128), pl.ds(a * FACTOR, FACTOR)] = yp


def _tc_retile(tabT):
  return pl.pallas_call(
      _tc_retile_body,
      grid=(NBLK,),
      in_specs=[pl.BlockSpec((FACTOR, RBLK), lambda j: (0, j))],
      out_specs=pl.BlockSpec((RBLK // 4, 128), lambda j: (j, 0)),
      out_shape=jax.ShapeDtypeStruct((MROWS, 128), jnp.float32),
  )(tabT)


def _sc_gather_body(idx_hbm, m_hbm, out_hbm, idx_v, buf_a, buf_b, sem):
  wid = lax.axis_index("s") * NUM_CORES + lax.axis_index("c")
  base = wid * BPW
  nch_tab = BPW // CHUNK
  for j in range(nch_tab):
    pltpu.sync_copy(idx_hbm.at[pl.ds(base + j * CHUNK, CHUNK)],
                    idx_v.at[j])
  for j in range(nch_tab):
    for v in range(CHUNK // LANES):
      sl = pl.ds(v * LANES, LANES)
      r = idx_v[j, sl]
      idx_v[j, sl] = (
          lax.shift_left(lax.shift_right_logical(r, 16), 14)
          + lax.shift_left(lax.shift_right_logical(r, 9) & 127, 7)
          + (r & 127))

  def gather(half_idx, buf):
    cps = []
    for j in range(NCH):
      cps.append(pltpu.async_copy(
          m_hbm.at[idx_v.at[half_idx * NCH + j]],
          buf.at[pl.ds(j * CHUNK, CHUNK)], sem))
    return cps

  def drain(cps):
    for c in cps:
      c.wait()

  cps = gather(0, buf_a)
  drain(cps)
  cps = gather(1, buf_b)
  pltpu.sync_copy(buf_a, out_hbm.at[pl.ds(base, HALF)])
  drain(cps)
  pltpu.sync_copy(buf_b, out_hbm.at[pl.ds(base + HALF, HALF)])


_sc_gather = functools.partial(
    pl.kernel,
    out_type=jax.ShapeDtypeStruct((BATCH, 128), jnp.float32),
    mesh=plsc.VectorSubcoreMesh(core_axis_name="c", subcore_axis_name="s"),
    scratch_types=[
        pltpu.VMEM((BPW // CHUNK, CHUNK), jnp.int32),
        pltpu.VMEM((HALF, 128), jnp.float32),
        pltpu.VMEM((HALF, 128), jnp.float32),
        pltpu.SemaphoreType.DMA,
    ],
)(_sc_gather_body)


TC_BLK = 2048


def _tc_affine_body(u_ref, i_ref, uidx_ref, iidx_ref, w_ref, b_ref, o_ref):
  qu = lax.shift_right_logical(uidx_ref[...], 7) & 3
  qi = lax.shift_right_logical(iidx_ref[...], 7) & 3
  u128 = u_ref[...]
  i128 = i_ref[...]
  uq = jnp.zeros((TC_BLK, FACTOR), jnp.float32)
  iq = jnp.zeros((TC_BLK, FACTOR), jnp.float32)
  for q in range(4):
    sl = slice(q * FACTOR, (q + 1) * FACTOR)
    uq = uq + jnp.where(qu == q, u128[:, sl], 0.0)
    iq = iq + jnp.where(qi == q, i128[:, sl], 0.0)
  s = jnp.sum(uq * iq * w_ref[...], axis=1) + b_ref[0, 0]
  o_ref[...] = jax.nn.sigmoid(s)


def _tc_affine(u_rows, i_rows, uidx, iidx, affine_w, affine_b):
  grid = (BATCH // TC_BLK,)
  return pl.pallas_call(
      _tc_affine_body,
      grid=grid,
      in_specs=[
          pl.BlockSpec((TC_BLK, 128), lambda i: (i, 0)),
          pl.BlockSpec((TC_BLK, 128), lambda i: (i, 0)),
          pl.BlockSpec((TC_BLK, 1), lambda i: (i, 0)),
          pl.BlockSpec((TC_BLK, 1), lambda i: (i, 0)),
          pl.BlockSpec((1, FACTOR), lambda i: (0, 0)),
          pl.BlockSpec(memory_space=pltpu.SMEM),
      ],
      out_specs=pl.BlockSpec((TC_BLK,), lambda i: (i,)),
      out_shape=jax.ShapeDtypeStruct((BATCH,), jnp.float32),
  )(u_rows, i_rows, uidx.reshape(BATCH, 1), iidx.reshape(BATCH, 1),
    affine_w, affine_b.reshape(1, 1))


def kernel(user_indices, item_indices, user_table, item_table,
           affine_w, affine_b):
  uidx = user_indices.astype(jnp.int32)
  iidx = item_indices.astype(jnp.int32)
  um = _tc_retile(jnp.transpose(user_table))
  u_rows = _sc_gather(uidx, um)
  im = _tc_retile(jnp.transpose(item_table))
  i_rows = _sc_gather(iidx, im)
  return _tc_affine(u_rows, i_rows, uidx, iidx, affine_w, affine_b)
